# baseline (speedup 1.0000x reference)
import functools

import jax
import jax.numpy as jnp
from jax import lax
from jax.experimental import pallas as pl
from jax.experimental.pallas import tpu as pltpu

D_MODEL = 1024
MAXLEN = 8192
TOTAL = 4 * 8192
R = 512
GRID = TOTAL // R
UNROLL = 8


def _tc_body(idx_ref, table_ref, out_ref):
    def body(j, carry):
        base = j * UNROLL
        for u in range(UNROLL):
            r = base + u
            out_ref[r] = table_ref[idx_ref[0, 0, r]]
        return carry

    lax.fori_loop(0, R // UNROLL, body, 0)


@jax.jit
def tc_gather(position_ids, pe):
    idx = position_ids.reshape(GRID, 1, R).astype(jnp.int32)
    table = pe.reshape(MAXLEN, 8, 128)
    out = pl.pallas_call(
        _tc_body,
        grid=(GRID,),
        in_specs=[
            pl.BlockSpec((1, 1, R), lambda i: (i, 0, 0), memory_space=pltpu.SMEM),
            pl.BlockSpec((MAXLEN, 8, 128), lambda i: (0, 0, 0)),
        ],
        out_specs=pl.BlockSpec((R, 8, 128), lambda i: (i, 0, 0)),
        out_shape=jax.ShapeDtypeStruct((TOTAL, 8, 128), jnp.float32),
    )(idx, table)
    return out.reshape(position_ids.shape + (D_MODEL,))


def kernel(position_ids, pe):
    return tc_gather(position_ids, pe)
